# baseline (speedup 1.0000x reference)
import functools

import jax
import jax.numpy as jnp
from jax import lax
from jax.experimental import pallas as pl
from jax.experimental.pallas import tpu as pltpu
from jax.experimental.pallas import tpu_sc as plsc

SEQ = 4096
B = 4
D = 2048
LANES = 16

_mesh = plsc.VectorSubcoreMesh(core_axis_name="c", subcore_axis_name="s")


@functools.partial(
    pl.kernel,
    mesh=_mesh,
    out_type=jax.ShapeDtypeStruct((B, D), jnp.float32),
    scratch_types=[
        pltpu.VMEM((LANES,), jnp.int32),
        pltpu.VMEM((LANES,), jnp.int32),
        pltpu.VMEM((LANES, D), jnp.float32),
        pltpu.SemaphoreType.DMA,
    ],
)
def _gather_rows(table_hbm, wp_hbm, out_hbm, wp_v, idx_v, rows_v, sem):
    c = lax.axis_index("c")
    s = lax.axis_index("s")

    @pl.when(jnp.logical_and(c == 0, s == 0))
    def _():
        pltpu.sync_copy(wp_hbm, wp_v)
        lane = lax.iota(jnp.int32, 16)
        wp = wp_v[...]
        idx = jnp.where(lane < B, wp * B + lane, 0)
        idx_v[...] = idx
        pltpu.async_copy(table_hbm.at[idx_v], rows_v, sem).wait()
        pltpu.sync_copy(rows_v.at[pl.ds(0, B)], out_hbm)


def kernel(src, word_pos):
    table = src.reshape(SEQ * B, D)
    wp = word_pos.astype(jnp.int32)
    wp_pad = jnp.concatenate([wp, jnp.zeros((LANES - B,), jnp.int32)])
    return _gather_rows(table, wp_pad)


# speedup vs baseline: 1.0167x; 1.0167x over previous
import functools

import jax
import jax.numpy as jnp
from jax import lax
from jax.experimental import pallas as pl
from jax.experimental.pallas import tpu as pltpu
from jax.experimental.pallas import tpu_sc as plsc

SEQ = 4096
B = 4
D = 2048
LANES = 16

_mesh = plsc.VectorSubcoreMesh(core_axis_name="c", subcore_axis_name="s",
                               num_cores=1)


@functools.partial(
    pl.kernel,
    mesh=_mesh,
    out_type=jax.ShapeDtypeStruct((B, D), jnp.float32),
    scratch_types=[
        pltpu.VMEM((LANES,), jnp.int32),
        pltpu.VMEM((LANES,), jnp.int32),
        pltpu.VMEM((LANES, D), jnp.float32),
        pltpu.SemaphoreType.DMA,
    ],
)
def _gather_rows(table_hbm, wp_hbm, out_hbm, wp_v, idx_v, rows_v, sem):
    c = lax.axis_index("c")
    s = lax.axis_index("s")

    @pl.when(jnp.logical_and(c == 0, s == 0))
    def _():
        pltpu.sync_copy(wp_hbm, wp_v)
        lane = lax.iota(jnp.int32, 16)
        wp = wp_v[...]
        idx = jnp.where(lane < B, wp * B + lane, 0)
        idx_v[...] = idx
        pltpu.async_copy(table_hbm.at[idx_v], rows_v, sem).wait()
        pltpu.sync_copy(rows_v.at[pl.ds(0, B)], out_hbm)


def kernel(src, word_pos):
    table = src.reshape(SEQ * B, D)
    wp = word_pos.astype(jnp.int32)
    wp_pad = jnp.concatenate([wp, jnp.zeros((LANES - B,), jnp.int32)])
    return _gather_rows(table, wp_pad)


# speedup vs baseline: 55.4103x; 54.4979x over previous
import jax
import jax.numpy as jnp
from jax.experimental import pallas as pl
from jax.experimental.pallas import tpu as pltpu

SEQ = 4096
B = 4
D = 2048


def _copy_body(idx_ref, src_ref, out_ref):
    del idx_ref
    b = pl.program_id(0)
    out_ref[pl.ds(b, 1), :] = src_ref[0, pl.ds(b, 1), :]


_grid_spec = pltpu.PrefetchScalarGridSpec(
    num_scalar_prefetch=1,
    grid=(B,),
    in_specs=[
        pl.BlockSpec((1, B, D), lambda b, idx_ref: (idx_ref[b], 0, 0)),
    ],
    out_specs=pl.BlockSpec((B, D), lambda b, idx_ref: (0, 0)),
)


def kernel(src, word_pos):
    idx = word_pos.astype(jnp.int32)
    return pl.pallas_call(
        _copy_body,
        grid_spec=_grid_spec,
        out_shape=jax.ShapeDtypeStruct((B, D), jnp.float32),
    )(idx, src)


# speedup vs baseline: 79.8177x; 1.4405x over previous
import jax
import jax.numpy as jnp
from jax.experimental import pallas as pl
from jax.experimental.pallas import tpu as pltpu

SEQ = 4096
B = 4
D = 2048


def _gather_body(idx_ref, src_ref, out_ref, sem):
    copies = []
    for b in range(B):
        copies.append(
            pltpu.make_async_copy(
                src_ref.at[idx_ref[b], b], out_ref.at[b], sem
            )
        )
    for c in copies:
        c.start()
    for c in copies:
        c.wait()


_grid_spec = pltpu.PrefetchScalarGridSpec(
    num_scalar_prefetch=1,
    grid=(1,),
    in_specs=[pl.BlockSpec(memory_space=pl.ANY)],
    out_specs=pl.BlockSpec((B, D), lambda g, idx_ref: (0, 0)),
    scratch_shapes=[pltpu.SemaphoreType.DMA],
)


def kernel(src, word_pos):
    idx = word_pos.astype(jnp.int32)
    return pl.pallas_call(
        _gather_body,
        grid_spec=_grid_spec,
        out_shape=jax.ShapeDtypeStruct((B, D), jnp.float32),
    )(idx, src)
